# Initial kernel scaffold; baseline (speedup 1.0000x reference)
import jax
import jax.numpy as jnp
from jax.experimental import pallas as pl


def kernel(x, edge_index, W1, b1, W2, b2):
    raise NotImplementedError("write your pallas kernel here")


# speedup vs baseline: 15.7201x; 15.7201x over previous
import functools

import jax
import jax.numpy as jnp
from jax import lax
from jax.experimental import pallas as pl
from jax.experimental.pallas import tpu as pltpu
from jax.experimental.pallas import tpu_sc as plsc

N = 10000
E = 320000
D = 128

NC = 2
NS = 16
C = 128
NCHUNK = E // C
CH_PER_CORE = NCHUNK // NC
FULL_J = CH_PER_CORE // NS
TAIL = CH_PER_CORE - FULL_J * NS
NWB = 10
WBR = N // NWB
DW = 16

_mesh = plsc.VectorSubcoreMesh(
    core_axis_name="c", subcore_axis_name="s", num_cores=NC, num_subcores=NS)


def _zero_vmem_2d(ref, rows, width):
  z16 = jnp.zeros((16,), jnp.float32)

  def row(i, _):
    for k in range(width // 16):
      ref[i, pl.ds(k * 16, 16)] = z16
    return 0

  lax.fori_loop(0, rows, row, 0)


def _zero_spmem_rows(acc, zb, base, nrows):
  zrows = zb.shape[0]
  full = nrows // zrows
  rem = nrows - full * zrows
  for k in range(full):
    pltpu.sync_copy(zb, acc.at[pl.ds(base + k * zrows, zrows)])
  if rem:
    pltpu.sync_copy(zb.at[pl.ds(0, rem)],
                    acc.at[pl.ds(base + full * zrows, rem)])


@functools.partial(
    pl.kernel,
    out_type=jax.ShapeDtypeStruct((NC, N, DW), jnp.float32),
    mesh=_mesh,
    compiler_params=pltpu.CompilerParams(use_tc_tiling_on_sc=False),
    scratch_types=[
        pltpu.VMEM_SHARED((N, DW), jnp.float32),
        pltpu.VMEM((C, DW), jnp.float32),
        pltpu.VMEM((C, DW), jnp.float32),
        pltpu.VMEM((1, C), jnp.int32),
    ],
)
def _deg_kernel(dst2d, out, degacc, ones, zb, didx):
  c = lax.axis_index("c")
  s = lax.axis_index("s")

  one16 = jnp.ones((16,), jnp.float32)
  z16 = jnp.zeros((16,), jnp.float32)

  def fill(i, _):
    ones[i, pl.ds(0, 16)] = one16
    zb[i, pl.ds(0, 16)] = z16
    return 0

  lax.fori_loop(0, C, fill, 0)

  @pl.when(s < NWB)
  def _():
    _zero_spmem_rows(degacc, zb, s * WBR, WBR)

  plsc.subcore_barrier()

  def chunk(cid):
    pltpu.sync_copy(dst2d.at[cid], didx.at[0])
    pltpu.sync_copy(ones, degacc.at[didx.at[0]], add=True)

  cbase = c * CH_PER_CORE

  def body(j, _):
    chunk(cbase + j * NS + s)
    return 0

  lax.fori_loop(0, FULL_J, body, 0)

  @pl.when(s < TAIL)
  def _():
    chunk(cbase + FULL_J * NS + s)

  plsc.subcore_barrier()

  @pl.when(s < NWB)
  def _():
    pltpu.sync_copy(degacc.at[pl.ds(s * WBR, WBR)],
                    out.at[c, pl.ds(s * WBR, WBR)])


@functools.partial(
    pl.kernel,
    out_type=jax.ShapeDtypeStruct((NC, N, D), jnp.float32),
    mesh=_mesh,
    scratch_types=[
        pltpu.VMEM_SHARED((N, D), jnp.float32),
        pltpu.VMEM((C, D), jnp.float32),
        pltpu.VMEM((1, C), jnp.int32),
        pltpu.VMEM((1, C), jnp.int32),
        pltpu.VMEM((C, D), jnp.float32),
        pltpu.SemaphoreType.DMA,
    ],
)
def _scatter_kernel(g, src2d, dst2d, out, acc, rows, sidx, didx, zb, sem):
  c = lax.axis_index("c")
  s = lax.axis_index("s")

  _zero_vmem_2d(zb, C, D)

  @pl.when(s < NWB)
  def _():
    _zero_spmem_rows(acc, zb, s * WBR, WBR)

  plsc.subcore_barrier()

  def chunk(cid):
    pltpu.sync_copy(src2d.at[cid], sidx.at[0])
    pltpu.sync_copy(dst2d.at[cid], didx.at[0])
    pltpu.async_copy(g.at[sidx.at[0]], rows, sem).wait()
    pltpu.sync_copy(rows, acc.at[didx.at[0]], add=True)

  cbase = c * CH_PER_CORE

  def body(j, _):
    chunk(cbase + j * NS + s)
    return 0

  lax.fori_loop(0, FULL_J, body, 0)

  @pl.when(s < TAIL)
  def _():
    chunk(cbase + FULL_J * NS + s)

  plsc.subcore_barrier()

  @pl.when(s < NWB)
  def _():
    pltpu.sync_copy(acc.at[pl.ds(s * WBR, WBR)],
                    out.at[c, pl.ds(s * WBR, WBR)])


_RB = 2000


def _dinv_block(da_ref, db_ref):
  deg = da_ref[:, 0:1] + db_ref[:, 0:1] + 1.0
  return lax.rsqrt(deg)


def _k1_body(x_ref, w_ref, da_ref, db_ref, o_ref):
  dinv = _dinv_block(da_ref, db_ref)
  h = jnp.dot(x_ref[...], w_ref[...], preferred_element_type=jnp.float32)
  o_ref[...] = h * dinv


def _k2_body(a0_ref, a1_ref, g_ref, b_ref, w_ref, da_ref, db_ref, o_ref):
  dinv = _dinv_block(da_ref, db_ref)
  t = (a0_ref[...] + a1_ref[...] + g_ref[...]) * dinv + b_ref[...]
  z = jnp.maximum(t, 0.0)
  h = jnp.dot(z, w_ref[...], preferred_element_type=jnp.float32)
  o_ref[...] = h * dinv


def _k3_body(a0_ref, a1_ref, g_ref, b_ref, da_ref, db_ref, o_ref):
  dinv = _dinv_block(da_ref, db_ref)
  o_ref[...] = (a0_ref[...] + a1_ref[...] + g_ref[...]) * dinv + b_ref[...]


def _row_spec(width):
  return pl.BlockSpec((_RB, width), lambda i: (i, 0))


def _full_spec(shape):
  return pl.BlockSpec(shape, lambda i: (0,) * len(shape))


def _tc_call(body, in_specs, n_out_width=D):
  return pl.pallas_call(
      body,
      grid=(N // _RB,),
      in_specs=in_specs,
      out_specs=_row_spec(n_out_width),
      out_shape=jax.ShapeDtypeStruct((N, n_out_width), jnp.float32),
  )


def kernel(x, edge_index, W1, b1, W2, b2):
  src2d = edge_index[0].reshape(NCHUNK, C)
  dst2d = edge_index[1].reshape(NCHUNK, C)

  degp = _deg_kernel(dst2d)
  dega, degb = degp[0], degp[1]

  g1 = _tc_call(
      _k1_body,
      [_row_spec(D), _full_spec((D, D)), _row_spec(DW), _row_spec(DW)],
  )(x, W1, dega, degb)

  s1 = _scatter_kernel(g1, src2d, dst2d)

  b1r = b1.reshape(1, D)
  b2r = b2.reshape(1, D)

  g2 = _tc_call(
      _k2_body,
      [_row_spec(D), _row_spec(D), _row_spec(D), _full_spec((1, D)),
       _full_spec((D, D)), _row_spec(DW), _row_spec(DW)],
  )(s1[0], s1[1], g1, b1r, W2, dega, degb)

  s2 = _scatter_kernel(g2, src2d, dst2d)

  out = _tc_call(
      _k3_body,
      [_row_spec(D), _row_spec(D), _row_spec(D), _full_spec((1, D)),
       _row_spec(DW), _row_spec(DW)],
  )(s2[0], s2[1], g2, b2r, dega, degb)

  return out


# speedup vs baseline: 27.8909x; 1.7742x over previous
import functools

import jax
import jax.numpy as jnp
from jax import lax
from jax.experimental import pallas as pl
from jax.experimental.pallas import tpu as pltpu
from jax.experimental.pallas import tpu_sc as plsc

N = 10000
E = 320000
D = 128

NC = 2
NS = 16
C = 128
NCHUNK = E // C
CH_PER_CORE = NCHUNK // NC
FULL_J = CH_PER_CORE // NS
TAIL = CH_PER_CORE - FULL_J * NS
NWB = 10
WBR = N // NWB
DW = 16

_mesh = plsc.VectorSubcoreMesh(
    core_axis_name="c", subcore_axis_name="s", num_cores=NC, num_subcores=NS)


def _zero_vmem_2d(ref, rows, width):
  z16 = jnp.zeros((16,), jnp.float32)

  def row(i, _):
    for k in range(width // 16):
      ref[i, pl.ds(k * 16, 16)] = z16
    return 0

  lax.fori_loop(0, rows, row, 0)


def _zero_spmem_rows(acc, zb, base, nrows):
  zrows = zb.shape[0]
  full = nrows // zrows
  rem = nrows - full * zrows
  for k in range(full):
    pltpu.sync_copy(zb, acc.at[pl.ds(base + k * zrows, zrows)])
  if rem:
    pltpu.sync_copy(zb.at[pl.ds(0, rem)],
                    acc.at[pl.ds(base + full * zrows, rem)])


@functools.partial(
    pl.kernel,
    out_type=jax.ShapeDtypeStruct((NC, N, DW), jnp.float32),
    mesh=_mesh,
    compiler_params=pltpu.CompilerParams(use_tc_tiling_on_sc=False),
    scratch_types=[
        pltpu.VMEM_SHARED((N, DW), jnp.float32),
        pltpu.VMEM((C, DW), jnp.float32),
        pltpu.VMEM((C, DW), jnp.float32),
        pltpu.VMEM((1, C), jnp.int32),
    ],
)
def _deg_kernel(dst2d, out, degacc, ones, zb, didx):
  c = lax.axis_index("c")
  s = lax.axis_index("s")

  one16 = jnp.ones((16,), jnp.float32)
  z16 = jnp.zeros((16,), jnp.float32)

  def fill(i, _):
    ones[i, pl.ds(0, 16)] = one16
    zb[i, pl.ds(0, 16)] = z16
    return 0

  lax.fori_loop(0, C, fill, 0)

  @pl.when(s < NWB)
  def _():
    _zero_spmem_rows(degacc, zb, s * WBR, WBR)

  plsc.subcore_barrier()

  def chunk(cid):
    pltpu.sync_copy(dst2d.at[cid], didx.at[0])
    pltpu.sync_copy(ones, degacc.at[didx.at[0]], add=True)

  cbase = c * CH_PER_CORE

  def body(j, _):
    chunk(cbase + j * NS + s)
    return 0

  lax.fori_loop(0, FULL_J, body, 0)

  @pl.when(s < TAIL)
  def _():
    chunk(cbase + FULL_J * NS + s)

  plsc.subcore_barrier()

  @pl.when(s < NWB)
  def _():
    pltpu.sync_copy(degacc.at[pl.ds(s * WBR, WBR)],
                    out.at[c, pl.ds(s * WBR, WBR)])


CPW = 80
CPW_LAST = NCHUNK - CPW * (NC * NS - 1)
NSLOT = 4


@functools.partial(
    pl.kernel,
    out_type=jax.ShapeDtypeStruct((NC, N, D), jnp.float32),
    mesh=_mesh,
    scratch_types=[
        pltpu.VMEM_SHARED((N, D), jnp.float32),
        pltpu.VMEM((2, C, D), jnp.float32),
        pltpu.VMEM((NSLOT, C), jnp.int32),
        pltpu.VMEM((NSLOT, C), jnp.int32),
        pltpu.SemaphoreType.DMA,
        pltpu.SemaphoreType.DMA,
        pltpu.SemaphoreType.DMA,
        pltpu.SemaphoreType.DMA,
        pltpu.SemaphoreType.DMA,
        pltpu.SemaphoreType.DMA,
    ],
)
def _scatter_kernel(g, src2d, dst2d, out, acc, rows, sidx, didx,
                    isem0, isem1, isem2, isem3, gsem0, gsem1):
  c = lax.axis_index("c")
  s = lax.axis_index("s")
  w = c * NS + s
  start = w * CPW
  cnt = jnp.where(w == NC * NS - 1, CPW_LAST, CPW)

  isems = (isem0, isem1, isem2, isem3)
  gsems = (gsem0, gsem1)

  def istart(j, k):
    pltpu.async_copy(src2d.at[start + j], sidx.at[k], isems[k])
    pltpu.async_copy(dst2d.at[start + j], didx.at[k], isems[k])

  def iwait(j, k):
    pltpu.make_async_copy(src2d.at[start + j], sidx.at[k], isems[k]).wait()
    pltpu.make_async_copy(dst2d.at[start + j], didx.at[k], isems[k]).wait()

  def gstart(k, b):
    pltpu.async_copy(g.at[sidx.at[k]], rows.at[b], gsems[b])

  def gwait(k, b):
    pltpu.make_async_copy(g.at[sidx.at[k]], rows.at[b], gsems[b]).wait()

  def ssync(k, b):
    pltpu.sync_copy(rows.at[b], acc.at[didx.at[k]], add=True)

  for k in range(NSLOT):
    istart(k, k)

  z16 = jnp.zeros((16,), jnp.float32)

  def zrow(i, _):
    for k in range(D // 16):
      rows[0, i, pl.ds(k * 16, 16)] = z16
    return 0

  lax.fori_loop(0, C, zrow, 0)

  @pl.when(s < NWB)
  def _():
    _zero_spmem_rows(acc, rows.at[0], s * WBR, WBR)

  plsc.subcore_barrier()

  iwait(0, 0)
  gstart(0, 0)

  def body(i, _):
    q = 4 * i
    iwait(q + 1, 1)
    gstart(1, 1)
    gwait(0, 0)
    ssync(0, 0)

    @pl.when(q + 4 < cnt)
    def _():
      istart(q + 4, 0)

    iwait(q + 2, 2)
    gstart(2, 0)
    gwait(1, 1)
    ssync(1, 1)

    @pl.when(q + 5 < cnt)
    def _():
      istart(q + 5, 1)

    iwait(q + 3, 3)
    gstart(3, 1)
    gwait(2, 0)
    ssync(2, 0)

    @pl.when(q + 6 < cnt)
    def _():
      istart(q + 6, 2)

    @pl.when(q + 4 < cnt)
    def _():
      iwait(q + 4, 0)
      gstart(0, 0)

    gwait(3, 1)
    ssync(3, 1)

    @pl.when(q + 7 < cnt)
    def _():
      istart(q + 7, 3)

    return 0

  lax.fori_loop(0, cnt // 4, body, 0)

  plsc.subcore_barrier()

  @pl.when(s < NWB)
  def _():
    pltpu.sync_copy(acc.at[pl.ds(s * WBR, WBR)],
                    out.at[c, pl.ds(s * WBR, WBR)])


_RB = 2000


def _dinv_block(da_ref, db_ref):
  deg = da_ref[:, 0:1] + db_ref[:, 0:1] + 1.0
  return lax.rsqrt(deg)


def _k1_body(x_ref, w_ref, da_ref, db_ref, o_ref):
  dinv = _dinv_block(da_ref, db_ref)
  h = jnp.dot(x_ref[...], w_ref[...], preferred_element_type=jnp.float32)
  o_ref[...] = h * dinv


def _k2_body(a0_ref, a1_ref, g_ref, b_ref, w_ref, da_ref, db_ref, o_ref):
  dinv = _dinv_block(da_ref, db_ref)
  t = (a0_ref[...] + a1_ref[...] + g_ref[...]) * dinv + b_ref[...]
  z = jnp.maximum(t, 0.0)
  h = jnp.dot(z, w_ref[...], preferred_element_type=jnp.float32)
  o_ref[...] = h * dinv


def _k3_body(a0_ref, a1_ref, g_ref, b_ref, da_ref, db_ref, o_ref):
  dinv = _dinv_block(da_ref, db_ref)
  o_ref[...] = (a0_ref[...] + a1_ref[...] + g_ref[...]) * dinv + b_ref[...]


def _row_spec(width):
  return pl.BlockSpec((_RB, width), lambda i: (i, 0))


def _full_spec(shape):
  return pl.BlockSpec(shape, lambda i: (0,) * len(shape))


def _tc_call(body, in_specs, n_out_width=D):
  return pl.pallas_call(
      body,
      grid=(N // _RB,),
      in_specs=in_specs,
      out_specs=_row_spec(n_out_width),
      out_shape=jax.ShapeDtypeStruct((N, n_out_width), jnp.float32),
  )


def kernel(x, edge_index, W1, b1, W2, b2):
  src2d = edge_index[0].reshape(NCHUNK, C)
  dst2d = edge_index[1].reshape(NCHUNK, C)

  degp = _deg_kernel(dst2d)
  dega, degb = degp[0], degp[1]

  g1 = _tc_call(
      _k1_body,
      [_row_spec(D), _full_spec((D, D)), _row_spec(DW), _row_spec(DW)],
  )(x, W1, dega, degb)

  s1 = _scatter_kernel(g1, src2d, dst2d)

  b1r = b1.reshape(1, D)
  b2r = b2.reshape(1, D)

  g2 = _tc_call(
      _k2_body,
      [_row_spec(D), _row_spec(D), _row_spec(D), _full_spec((1, D)),
       _full_spec((D, D)), _row_spec(DW), _row_spec(DW)],
  )(s1[0], s1[1], g1, b1r, W2, dega, degb)

  s2 = _scatter_kernel(g2, src2d, dst2d)

  out = _tc_call(
      _k3_body,
      [_row_spec(D), _row_spec(D), _row_spec(D), _full_spec((1, D)),
       _row_spec(DW), _row_spec(DW)],
  )(s2[0], s2[1], g2, b2r, dega, degb)

  return out
